# Initial kernel scaffold; baseline (speedup 1.0000x reference)
import jax
import jax.numpy as jnp
from jax.experimental import pallas as pl


def kernel(k_val, v_val, k_cache, v_cache):
    raise NotImplementedError("write your pallas kernel here")


# speedup vs baseline: 8.4218x; 8.4218x over previous
import jax
import jax.numpy as jnp
from jax.experimental import pallas as pl


def _copy_kernel(k_ref, v_ref, k_out_ref, v_out_ref):
    k_out_ref[...] = k_ref[...]
    v_out_ref[...] = v_ref[...]


def kernel(k_val, v_val, k_cache, v_cache):
    del k_cache, v_cache
    b, s, h, d = k_val.shape
    k2 = k_val.reshape(b, s, h * d)
    v2 = v_val.reshape(b, s, h * d)
    k_out, v_out = pl.pallas_call(
        _copy_kernel,
        grid=(b,),
        in_specs=[
            pl.BlockSpec((1, s, h * d), lambda i: (i, 0, 0)),
            pl.BlockSpec((1, s, h * d), lambda i: (i, 0, 0)),
        ],
        out_specs=[
            pl.BlockSpec((1, s, h * d), lambda i: (i, 0, 0)),
            pl.BlockSpec((1, s, h * d), lambda i: (i, 0, 0)),
        ],
        out_shape=[
            jax.ShapeDtypeStruct((b, s, h * d), k_val.dtype),
            jax.ShapeDtypeStruct((b, s, h * d), v_val.dtype),
        ],
    )(k2, v2)
    return (k_out.reshape(b, s, h, d), v_out.reshape(b, s, h, d))


# speedup vs baseline: 9.8657x; 1.1714x over previous
import jax
import jax.numpy as jnp
from jax.experimental import pallas as pl


def _copy_kernel(k_ref, v_ref, k_out_ref, v_out_ref):
    k_out_ref[...] = k_ref[...]
    v_out_ref[...] = v_ref[...]


def kernel(k_val, v_val, k_cache, v_cache):
    del k_cache, v_cache
    b, s, h, d = k_val.shape
    rows = b * s
    cols = h * d
    blk = 64
    k2 = k_val.reshape(rows, cols)
    v2 = v_val.reshape(rows, cols)
    k_out, v_out = pl.pallas_call(
        _copy_kernel,
        grid=(rows // blk,),
        in_specs=[
            pl.BlockSpec((blk, cols), lambda i: (i, 0)),
            pl.BlockSpec((blk, cols), lambda i: (i, 0)),
        ],
        out_specs=[
            pl.BlockSpec((blk, cols), lambda i: (i, 0)),
            pl.BlockSpec((blk, cols), lambda i: (i, 0)),
        ],
        out_shape=[
            jax.ShapeDtypeStruct((rows, cols), k_val.dtype),
            jax.ShapeDtypeStruct((rows, cols), v_val.dtype),
        ],
    )(k2, v2)
    return (k_out.reshape(b, s, h, d), v_out.reshape(b, s, h, d))


# speedup vs baseline: 10.8247x; 1.0972x over previous
import jax
import jax.numpy as jnp
from jax.experimental import pallas as pl


def _copy_kernel(k_ref, v_ref, k_out_ref, v_out_ref):
    k_out_ref[...] = k_ref[...]
    v_out_ref[...] = v_ref[...]


def kernel(k_val, v_val, k_cache, v_cache):
    del k_cache, v_cache
    b, s, h, d = k_val.shape
    rows = b * s
    cols = h * d
    blk = 128
    k2 = k_val.reshape(rows, cols)
    v2 = v_val.reshape(rows, cols)
    k_out, v_out = pl.pallas_call(
        _copy_kernel,
        grid=(rows // blk,),
        in_specs=[
            pl.BlockSpec((blk, cols), lambda i: (i, 0)),
            pl.BlockSpec((blk, cols), lambda i: (i, 0)),
        ],
        out_specs=[
            pl.BlockSpec((blk, cols), lambda i: (i, 0)),
            pl.BlockSpec((blk, cols), lambda i: (i, 0)),
        ],
        out_shape=[
            jax.ShapeDtypeStruct((rows, cols), k_val.dtype),
            jax.ShapeDtypeStruct((rows, cols), v_val.dtype),
        ],
    )(k2, v2)
    return (k_out.reshape(b, s, h, d), v_out.reshape(b, s, h, d))


# speedup vs baseline: 11.5247x; 1.0647x over previous
import jax
import jax.numpy as jnp
from jax.experimental import pallas as pl


def _copy_kernel(k_ref, v_ref, k_out_ref, v_out_ref):
    k_out_ref[...] = k_ref[...]
    v_out_ref[...] = v_ref[...]


def kernel(k_val, v_val, k_cache, v_cache):
    del k_cache, v_cache
    b, s, h, d = k_val.shape
    rows = b * s
    cols = h * d
    blk = 256
    k2 = k_val.reshape(rows, cols)
    v2 = v_val.reshape(rows, cols)
    k_out, v_out = pl.pallas_call(
        _copy_kernel,
        grid=(rows // blk,),
        in_specs=[
            pl.BlockSpec((blk, cols), lambda i: (i, 0)),
            pl.BlockSpec((blk, cols), lambda i: (i, 0)),
        ],
        out_specs=[
            pl.BlockSpec((blk, cols), lambda i: (i, 0)),
            pl.BlockSpec((blk, cols), lambda i: (i, 0)),
        ],
        out_shape=[
            jax.ShapeDtypeStruct((rows, cols), k_val.dtype),
            jax.ShapeDtypeStruct((rows, cols), v_val.dtype),
        ],
    )(k2, v2)
    return (k_out.reshape(b, s, h, d), v_out.reshape(b, s, h, d))
